# baseline (speedup 1.0000x reference)
import functools

import jax
import jax.numpy as jnp
from jax import lax
from jax.experimental import pallas as pl
from jax.experimental.pallas import tpu as pltpu
from jax.experimental.pallas import tpu_sc as plsc

NUM_UNITS = 16
NUM_TACTICS = 16
ACTION_DIM = 32
OUT_W = 128
_SC_CORES = 2
_SC_SUBCORES = 16
_SC_WORKERS = _SC_CORES * _SC_SUBCORES
_CHUNK = 128


def _dot(a, b, precision=None):
    return lax.dot_general(a, b, (((1,), (0,)), ((), ())),
                           preferred_element_type=jnp.float32,
                           precision=precision)


def _sort_body(uoh_ref, pos_ref, offs_ref, *, N, C):
    uoh = uoh_ref[...]
    mx = jnp.max(uoh, axis=1, keepdims=True)
    lanes = lax.broadcasted_iota(jnp.int32, (N, NUM_UNITS), 1)
    idx = jnp.min(jnp.where(uoh == mx, lanes, NUM_UNITS), axis=1,
                  keepdims=True)
    oh = (idx == lanes).astype(jnp.bfloat16)

    r = lax.broadcasted_iota(jnp.int32, (C, C), 0)
    c = lax.broadcasted_iota(jnp.int32, (C, C), 1)
    L = (r >= c).astype(jnp.bfloat16)

    cum = jnp.zeros((1, NUM_UNITS), jnp.float32)
    parts = []
    for ci in range(N // C):
        oh_c = oh[ci * C:(ci + 1) * C]
        incl = _dot(L, oh_c)
        cnt_c = incl[C - 1:C, :]
        excl = incl - oh_c.astype(jnp.float32)
        parts.append(excl + cum)
        cum = cum + cnt_c
    part = jnp.concatenate(parts, axis=0)

    u1 = lax.broadcasted_iota(jnp.int32, (NUM_UNITS, NUM_UNITS), 0)
    u2 = lax.broadcasted_iota(jnp.int32, (NUM_UNITS, NUM_UNITS), 1)
    T = (u1 < u2).astype(jnp.float32)
    offs = _dot(cum, T, precision=lax.Precision.HIGHEST)

    posf = jnp.sum(oh.astype(jnp.float32) * (part + offs), axis=1,
                   keepdims=True)
    pos_ref[...] = posf.astype(jnp.int32)
    offs_ref[...] = jnp.broadcast_to(offs.astype(jnp.int32), (8, NUM_UNITS))


def _sort_call(uoh, *, N, interpret=False):
    return pl.pallas_call(
        functools.partial(_sort_body, N=N, C=256),
        out_shape=[jax.ShapeDtypeStruct((N, 1), jnp.int32),
                   jax.ShapeDtypeStruct((8, NUM_UNITS), jnp.int32)],
        interpret=interpret,
    )(uoh)


def _route_in(obs, tacpad, pos1d):
    N = obs.shape[0]
    rows_per_w = N // _SC_WORKERS
    nch = rows_per_w // _CHUNK
    mesh = plsc.VectorSubcoreMesh(core_axis_name="c", subcore_axis_name="s")

    @functools.partial(
        pl.kernel, mesh=mesh,
        out_type=[jax.ShapeDtypeStruct((N, 128), jnp.float32),
                  jax.ShapeDtypeStruct((N, 128), jnp.float32)],
        scratch_types=[pltpu.VMEM((_CHUNK,), jnp.int32),
                       pltpu.VMEM((_CHUNK, 128), jnp.float32),
                       pltpu.VMEM((_CHUNK, 128), jnp.float32),
                       pltpu.SemaphoreType.DMA],
    )
    def k(obs_hbm, tac_hbm, pos_hbm, obs_out, tac_out, idx_v, obuf, tbuf, sem):
        wid = lax.axis_index("s") * _SC_CORES + lax.axis_index("c")
        base = wid * rows_per_w
        for j in range(nch):
            r0 = base + j * _CHUNK
            pltpu.sync_copy(pos_hbm.at[pl.ds(r0, _CHUNK)], idx_v)
            pltpu.sync_copy(obs_hbm.at[pl.ds(r0, _CHUNK)], obuf)
            pltpu.async_copy(obuf, obs_out.at[idx_v], sem).wait()
            pltpu.sync_copy(tac_hbm.at[pl.ds(r0, _CHUNK)], tbuf)
            pltpu.async_copy(tbuf, tac_out.at[idx_v], sem).wait()

    return k(obs, tacpad, pos1d)


def _route_out(comb_s, pos1d):
    N = comb_s.shape[0]
    rows_per_w = N // _SC_WORKERS
    nch = rows_per_w // _CHUNK
    mesh = plsc.VectorSubcoreMesh(core_axis_name="c", subcore_axis_name="s")

    @functools.partial(
        pl.kernel, mesh=mesh,
        out_type=jax.ShapeDtypeStruct((N, OUT_W), jnp.float32),
        scratch_types=[pltpu.VMEM((_CHUNK,), jnp.int32),
                       pltpu.VMEM((_CHUNK, OUT_W), jnp.float32),
                       pltpu.SemaphoreType.DMA],
    )
    def k(comb_hbm, pos_hbm, out_hbm, idx_v, buf, sem):
        wid = lax.axis_index("s") * _SC_CORES + lax.axis_index("c")
        base = wid * rows_per_w
        for j in range(nch):
            r0 = base + j * _CHUNK
            pltpu.sync_copy(pos_hbm.at[pl.ds(r0, _CHUNK)], idx_v)
            pltpu.async_copy(comb_hbm.at[idx_v], buf, sem).wait()
            pltpu.sync_copy(buf, out_hbm.at[pl.ds(r0, _CHUNK)])

    return k(comb_s, pos1d)


def _heads_body(obs_ref, tac_ref, offs_ref, W1a_ref, W1b_ref, b1_ref, W2_ref,
                b2_ref, Wh1_ref, bh1_ref, Wh2_ref, bh2_ref, comb_ref,
                *, B, N):
    h1 = _dot(obs_ref[...].astype(jnp.bfloat16), W1a_ref[...]) \
        + _dot(tac_ref[...].astype(jnp.bfloat16), W1b_ref[...])
    h1 = jnp.maximum(h1 + b1_ref[...], 0.0)
    h = jnp.maximum(_dot(h1.astype(jnp.bfloat16), W2_ref[...]) + b2_ref[...],
                    0.0)
    hb = h.astype(jnp.bfloat16)

    row0 = pl.program_id(0) * B
    jrow = lax.broadcasted_iota(jnp.int32, (B, 1), 0) + row0
    comb_ref[...] = jnp.zeros((B, OUT_W), jnp.float32)
    for u in range(NUM_UNITS):
        lo = offs_ref[0, u]
        hi = offs_ref[0, u + 1] if u < NUM_UNITS - 1 else N

        @pl.when(jnp.logical_and(lo < row0 + B, hi > row0))
        def _run(u=u, lo=lo, hi=hi):
            hv = jnp.maximum(_dot(hb, Wh1_ref[u]) + bh1_ref[u:u + 1], 0.0)
            out = _dot(hv.astype(jnp.bfloat16), Wh2_ref[u]) + bh2_ref[u:u + 1]
            m = jnp.logical_and(jrow >= lo, jrow < hi).astype(jnp.float32)
            comb_ref[...] += m * out


def _heads_call(obs_s, tac_s, offs, W1a, W1b, b1, W2, b2, Wh1, bh1, Wh2, bh2,
                *, block_rows, interpret=False):
    N = obs_s.shape[0]
    B = block_rows
    grid = (N // B,)

    def rows(i):
        return (i, 0)

    def full2(i):
        return (0, 0)

    def full3(i):
        return (0, 0, 0)

    return pl.pallas_call(
        functools.partial(_heads_body, B=B, N=N),
        grid=grid,
        in_specs=[
            pl.BlockSpec((B, 128), rows),
            pl.BlockSpec((B, 128), rows),
            pl.BlockSpec(offs.shape, full2),
            pl.BlockSpec(W1a.shape, full2),
            pl.BlockSpec(W1b.shape, full2),
            pl.BlockSpec(b1.shape, full2),
            pl.BlockSpec(W2.shape, full2),
            pl.BlockSpec(b2.shape, full2),
            pl.BlockSpec(Wh1.shape, full3),
            pl.BlockSpec(bh1.shape, full2),
            pl.BlockSpec(Wh2.shape, full3),
            pl.BlockSpec(bh2.shape, full2),
        ],
        out_specs=pl.BlockSpec((B, OUT_W), rows),
        out_shape=jax.ShapeDtypeStruct((N, OUT_W), jnp.float32),
        interpret=interpret,
    )(obs_s, tac_s, offs, W1a, W1b, b1, W2, b2, Wh1, bh1, Wh2, bh2)


def kernel(team_obs_rep, tactic_onehot_rep, unit_ids_onehot, W1, b1, W2, b2,
           pW1, pb1, pW2, pb2, vW1, vb1, vW2, vb2, *, block_rows=512,
           interpret=False):
    bf = jnp.bfloat16
    N = team_obs_rep.shape[0]

    pos, offs = _sort_call(unit_ids_onehot, N=N, interpret=interpret)
    pos1d = pos.reshape(N)

    tacpad = jnp.pad(tactic_onehot_rep, ((0, 0), (0, 128 - NUM_TACTICS)))
    obs_s, tac_s = _route_in(team_obs_rep, tacpad, pos1d)

    Wh1 = jnp.concatenate([pW1, vW1], axis=2).astype(bf)
    bh1 = jnp.concatenate([pb1, vb1], axis=1)
    z = jnp.zeros((NUM_UNITS, 128, 1), jnp.float32)
    top = jnp.concatenate([pW2] + [z] * (OUT_W - ACTION_DIM), axis=2)
    bot = jnp.concatenate([z] * ACTION_DIM + [vW2] + [z] * (OUT_W - ACTION_DIM - 1), axis=2)
    Wh2 = jnp.concatenate([top, bot], axis=1).astype(bf)
    bh2 = jnp.concatenate(
        [pb2, vb2, jnp.zeros((NUM_UNITS, OUT_W - ACTION_DIM - 1), jnp.float32)],
        axis=1)

    W1bp = jnp.pad(W1[128:], ((0, 128 - NUM_TACTICS), (0, 0)))
    comb_s = _heads_call(
        obs_s, tac_s, offs,
        W1[:128].astype(bf), W1bp.astype(bf), b1.reshape(1, -1),
        W2.astype(bf), b2.reshape(1, -1), Wh1, bh1, Wh2, bh2,
        block_rows=block_rows, interpret=interpret)

    comb = _route_out(comb_s, pos1d)
    return comb[:, :ACTION_DIM], comb[:, ACTION_DIM]


# speedup vs baseline: 1.3111x; 1.3111x over previous
import functools

import jax
import jax.numpy as jnp
from jax import lax
from jax.experimental import pallas as pl
from jax.experimental.pallas import tpu as pltpu
from jax.experimental.pallas import tpu_sc as plsc

NUM_UNITS = 16
NUM_TACTICS = 16
ACTION_DIM = 32
HEAD_HIDDEN = 128
TRUNK_HIDDEN = 256
OUT_W = 128
_SC_CORES = 2
_SC_SUBCORES = 16
_SC_WORKERS = _SC_CORES * _SC_SUBCORES
_CHUNK = 128


def _dot(a, b, precision=None):
    return lax.dot_general(a, b, (((1,), (0,)), ((), ())),
                           preferred_element_type=jnp.float32,
                           precision=precision)


def _sort_body(uoh_ref, pos_ref, offs_ref, *, N, C):
    uoh = uoh_ref[...]
    mx = jnp.max(uoh, axis=1, keepdims=True)
    lanes = lax.broadcasted_iota(jnp.int32, (N, NUM_UNITS), 1)
    idx = jnp.min(jnp.where(uoh == mx, lanes, NUM_UNITS), axis=1,
                  keepdims=True)
    oh = (idx == lanes).astype(jnp.bfloat16)

    r = lax.broadcasted_iota(jnp.int32, (C, C), 0)
    c = lax.broadcasted_iota(jnp.int32, (C, C), 1)
    L = (r >= c).astype(jnp.bfloat16)

    cum = jnp.zeros((1, NUM_UNITS), jnp.float32)
    parts = []
    for ci in range(N // C):
        oh_c = oh[ci * C:(ci + 1) * C]
        incl = _dot(L, oh_c)
        cnt_c = incl[C - 1:C, :]
        excl = incl - oh_c.astype(jnp.float32)
        parts.append(excl + cum)
        cum = cum + cnt_c
    part = jnp.concatenate(parts, axis=0)

    u1 = lax.broadcasted_iota(jnp.int32, (NUM_UNITS, NUM_UNITS), 0)
    u2 = lax.broadcasted_iota(jnp.int32, (NUM_UNITS, NUM_UNITS), 1)
    T = (u1 < u2).astype(jnp.float32)
    offs = _dot(cum, T, precision=lax.Precision.HIGHEST)

    posf = jnp.sum(oh.astype(jnp.float32) * (part + offs), axis=1,
                   keepdims=True)
    pos_ref[...] = posf.astype(jnp.int32)
    offs_ref[...] = jnp.broadcast_to(offs.astype(jnp.int32), (8, NUM_UNITS))


def _sort_call(uoh, *, N, interpret=False):
    return pl.pallas_call(
        functools.partial(_sort_body, N=N, C=256),
        out_shape=[jax.ShapeDtypeStruct((N, 1), jnp.int32),
                   jax.ShapeDtypeStruct((8, NUM_UNITS), jnp.int32)],
        interpret=interpret,
    )(uoh)


def _route_in(obs, tacpad, pos1d):
    N = obs.shape[0]
    rows_per_w = N // _SC_WORKERS
    nch = rows_per_w // _CHUNK
    mesh = plsc.VectorSubcoreMesh(core_axis_name="c", subcore_axis_name="s")

    @functools.partial(
        pl.kernel, mesh=mesh,
        out_type=[jax.ShapeDtypeStruct((N, 128), jnp.float32),
                  jax.ShapeDtypeStruct((N, 128), jnp.float32)],
        scratch_types=[pltpu.VMEM((_CHUNK,), jnp.int32),
                       pltpu.VMEM((_CHUNK, 128), jnp.float32),
                       pltpu.VMEM((_CHUNK, 128), jnp.float32),
                       pltpu.SemaphoreType.DMA],
    )
    def k(obs_hbm, tac_hbm, pos_hbm, obs_out, tac_out, idx_v, obuf, tbuf, sem):
        wid = lax.axis_index("s") * _SC_CORES + lax.axis_index("c")
        base = wid * rows_per_w
        for j in range(nch):
            r0 = base + j * _CHUNK
            pltpu.sync_copy(pos_hbm.at[pl.ds(r0, _CHUNK)], idx_v)
            pltpu.sync_copy(obs_hbm.at[pl.ds(r0, _CHUNK)], obuf)
            pltpu.async_copy(obuf, obs_out.at[idx_v], sem).wait()
            pltpu.sync_copy(tac_hbm.at[pl.ds(r0, _CHUNK)], tbuf)
            pltpu.async_copy(tbuf, tac_out.at[idx_v], sem).wait()

    return k(obs, tacpad, pos1d)


def _route_out(comb_s, pos1d):
    N = comb_s.shape[0]
    rows_per_w = N // _SC_WORKERS
    nch = rows_per_w // _CHUNK
    mesh = plsc.VectorSubcoreMesh(core_axis_name="c", subcore_axis_name="s")

    @functools.partial(
        pl.kernel, mesh=mesh,
        out_type=jax.ShapeDtypeStruct((N, OUT_W), jnp.float32),
        scratch_types=[pltpu.VMEM((_CHUNK,), jnp.int32),
                       pltpu.VMEM((_CHUNK, OUT_W), jnp.float32),
                       pltpu.SemaphoreType.DMA],
    )
    def k(comb_hbm, pos_hbm, out_hbm, idx_v, buf, sem):
        wid = lax.axis_index("s") * _SC_CORES + lax.axis_index("c")
        base = wid * rows_per_w
        for j in range(nch):
            r0 = base + j * _CHUNK
            pltpu.sync_copy(pos_hbm.at[pl.ds(r0, _CHUNK)], idx_v)
            pltpu.async_copy(comb_hbm.at[idx_v], buf, sem).wait()
            pltpu.sync_copy(buf, out_hbm.at[pl.ds(r0, _CHUNK)])

    return k(comb_s, pos1d)


def _heads_body(obs_ref, tac_ref, offs_ref, W1a_ref, W1b_ref, b1_ref, W2_ref,
                b2_ref, pW1_ref, pb1_ref, pW2_ref, pb2_ref, vW1_ref, vb1_ref,
                vW2_ref, vb2_ref, comb_ref, Wh1_s, bh1_s, Wh2_s, bh2_s,
                *, B, N):
    bf = jnp.bfloat16
    H2 = 2 * HEAD_HIDDEN

    @pl.when(pl.program_id(0) == 0)
    def _pack():
        lane = lax.broadcasted_iota(jnp.int32, (HEAD_HIDDEN, OUT_W), 1)
        lane1 = lax.broadcasted_iota(jnp.int32, (1, OUT_W), 1)
        zpad = jnp.zeros((HEAD_HIDDEN, OUT_W - ACTION_DIM), bf)
        zpadb = jnp.zeros((1, OUT_W - ACTION_DIM), jnp.float32)
        for u in range(NUM_UNITS):
            Wh1_s[u, :, 0:HEAD_HIDDEN] = pW1_ref[u].astype(bf)
            Wh1_s[u, :, HEAD_HIDDEN:H2] = vW1_ref[u].astype(bf)
            bh1_s[u:u + 1, 0:HEAD_HIDDEN] = pb1_ref[u:u + 1]
            bh1_s[u:u + 1, HEAD_HIDDEN:H2] = vb1_ref[u:u + 1]
            Wh2_s[u, 0:HEAD_HIDDEN, :] = jnp.concatenate(
                [pW2_ref[u].astype(bf), zpad], axis=1)
            vcol = jnp.broadcast_to(vW2_ref[u], (HEAD_HIDDEN, OUT_W))
            Wh2_s[u, HEAD_HIDDEN:H2, :] = jnp.where(
                lane == ACTION_DIM, vcol, 0.0).astype(bf)
            bh2_s[u:u + 1, :] = jnp.concatenate(
                [pb2_ref[u:u + 1], zpadb], axis=1) + jnp.where(
                    lane1 == ACTION_DIM,
                    jnp.broadcast_to(vb2_ref[u:u + 1], (1, OUT_W)), 0.0)

    h1 = _dot(obs_ref[...].astype(bf), W1a_ref[...]) \
        + _dot(tac_ref[...].astype(bf), W1b_ref[...])
    h1 = jnp.maximum(h1 + b1_ref[...], 0.0)
    h = jnp.maximum(_dot(h1.astype(bf), W2_ref[...]) + b2_ref[...], 0.0)
    hb = h.astype(bf)

    row0 = pl.program_id(0) * B
    jrow = lax.broadcasted_iota(jnp.int32, (B, 1), 0) + row0
    comb_ref[...] = jnp.zeros((B, OUT_W), jnp.float32)
    for u in range(NUM_UNITS):
        lo = offs_ref[0, u]
        hi = offs_ref[0, u + 1] if u < NUM_UNITS - 1 else N

        @pl.when(jnp.logical_and(lo < row0 + B, hi > row0))
        def _run(u=u, lo=lo, hi=hi):
            hv = jnp.maximum(_dot(hb, Wh1_s[u]) + bh1_s[u:u + 1], 0.0)
            out = _dot(hv.astype(bf), Wh2_s[u]) + bh2_s[u:u + 1]
            m = jnp.logical_and(jrow >= lo, jrow < hi).astype(jnp.float32)
            comb_ref[...] += m * out


def _heads_call(obs_s, tac_s, offs, W1a, W1b, b1, W2, b2, pW1, pb1, pW2, pb2,
                vW1, vb1, vW2, vb2, *, block_rows, interpret=False):
    N = obs_s.shape[0]
    B = block_rows
    grid = (N // B,)

    def rows(i):
        return (i, 0)

    def full2(i):
        return (0, 0)

    def full3(i):
        return (0, 0, 0)

    H2 = 2 * HEAD_HIDDEN
    return pl.pallas_call(
        functools.partial(_heads_body, B=B, N=N),
        grid=grid,
        in_specs=[
            pl.BlockSpec((B, 128), rows),
            pl.BlockSpec((B, 128), rows),
            pl.BlockSpec(offs.shape, full2),
            pl.BlockSpec(W1a.shape, full2),
            pl.BlockSpec(W1b.shape, full2),
            pl.BlockSpec(b1.shape, full2),
            pl.BlockSpec(W2.shape, full2),
            pl.BlockSpec(b2.shape, full2),
            pl.BlockSpec(pW1.shape, full3),
            pl.BlockSpec(pb1.shape, full2),
            pl.BlockSpec(pW2.shape, full3),
            pl.BlockSpec(pb2.shape, full2),
            pl.BlockSpec(vW1.shape, full3),
            pl.BlockSpec(vb1.shape, full2),
            pl.BlockSpec(vW2.shape, full3),
            pl.BlockSpec(vb2.shape, full2),
        ],
        out_specs=pl.BlockSpec((B, OUT_W), rows),
        out_shape=jax.ShapeDtypeStruct((N, OUT_W), jnp.float32),
        scratch_shapes=[
            pltpu.VMEM((NUM_UNITS, TRUNK_HIDDEN, H2), jnp.bfloat16),
            pltpu.VMEM((NUM_UNITS, H2), jnp.float32),
            pltpu.VMEM((NUM_UNITS, H2, OUT_W), jnp.bfloat16),
            pltpu.VMEM((NUM_UNITS, OUT_W), jnp.float32),
        ],
        interpret=interpret,
    )(obs_s, tac_s, offs, W1a, W1b, b1, W2, b2, pW1, pb1, pW2, pb2, vW1, vb1,
      vW2, vb2)


def kernel(team_obs_rep, tactic_onehot_rep, unit_ids_onehot, W1, b1, W2, b2,
           pW1, pb1, pW2, pb2, vW1, vb1, vW2, vb2, *, block_rows=2048,
           interpret=False):
    bf = jnp.bfloat16
    N = team_obs_rep.shape[0]

    pos, offs = _sort_call(unit_ids_onehot, N=N, interpret=interpret)
    pos1d = pos.reshape(N)

    tacpad = jnp.pad(tactic_onehot_rep, ((0, 0), (0, 128 - NUM_TACTICS)))
    obs_s, tac_s = _route_in(team_obs_rep, tacpad, pos1d)

    W1bp = jnp.pad(W1[128:], ((0, 128 - NUM_TACTICS), (0, 0)))
    comb_s = _heads_call(
        obs_s, tac_s, offs,
        W1[:128].astype(bf), W1bp.astype(bf), b1.reshape(1, -1),
        W2.astype(bf), b2.reshape(1, -1),
        pW1, pb1, pW2, pb2, vW1, vb1, vW2, vb2,
        block_rows=block_rows, interpret=interpret)

    comb = _route_out(comb_s, pos1d)
    return comb[:, :ACTION_DIM], comb[:, ACTION_DIM]
